# baseline (speedup 1.0000x reference)
import functools

import jax
import jax.numpy as jnp
import numpy as np
from jax import lax
from jax.experimental import pallas as pl
from jax.experimental.pallas import tpu as pltpu
from jax.experimental.pallas import tpu_sc as plsc

NC = 2
NS = 16
NW = NC * NS
L = 16
LN_EPS = 1e-5
BLK = 128
NBUF = 4


def _rsqrt_newton(x):
    i = lax.bitcast_convert_type(x, jnp.int32)
    i = jnp.int32(0x5F3759DF) - lax.shift_right_arithmetic(i, jnp.int32(1))
    y = lax.bitcast_convert_type(i, jnp.float32)
    half_x = jnp.float32(0.5) * x
    for _ in range(3):
        y = y * (jnp.float32(1.5) - half_x * y * y)
    return y


def _build(N, S, emb, interpret=False):
    FV = emb // L
    TI = emb // 8
    inv_emb = np.float32(1.0 / emb)
    nblk = N // BLK

    mesh = plsc.VectorSubcoreMesh(
        core_axis_name="c", subcore_axis_name="s", num_cores=NC, num_subcores=NS
    )

    @functools.partial(
        pl.kernel,
        out_type=jax.ShapeDtypeStruct((S, TI, nblk, 8, BLK), jnp.float32),
        mesh=mesh,
        scratch_types=[
            pltpu.VMEM((S, BLK), jnp.int32),
            pltpu.VMEM((S, BLK), jnp.int32),
            pltpu.VMEM((S, emb), jnp.float32),
            pltpu.VMEM((2, emb), jnp.float32),
            pltpu.VMEM((NBUF, BLK, emb), jnp.float32),
            pltpu.VMEM((NBUF, TI, 8, BLK + 1), jnp.float32),
            pltpu.VMEM((NBUF, 2, BLK), jnp.float32),
        ]
        + [pltpu.SemaphoreType.DMA] * (2 * NBUF),
        compiler_params=pltpu.CompilerParams(
            needs_layout_passes=False, use_tc_tiling_on_sc=False
        ),
        interpret=interpret,
    )
    def k(idx_hbm, table_hbm, pos_hbm, wb_hbm, out_hbm,
          idx_v, stage_v, pos_v, wb_v, x_v, out_v, st_v, *sems):
        gsems = sems[:NBUF]
        osems = sems[NBUF:]
        wid = lax.axis_index("s") * NC + lax.axis_index("c")

        pltpu.sync_copy(idx_hbm.at[wid], stage_v)
        pltpu.sync_copy(pos_hbm, pos_v)
        pltpu.sync_copy(wb_hbm, wb_v)

        tio = lax.iota(jnp.int32, L) * jnp.int32(S)

        @plsc.parallel_loop(0, S, unroll=2)
        def _(kk):
            for v in range(BLK // L):
                f = tio + (kk + jnp.int32(v * L * S))
                val = plsc.load_gather(
                    stage_v,
                    [lax.shift_right_arithmetic(f, jnp.int32(7)),
                     f & jnp.int32(BLK - 1)],
                )
                idx_v[kk, pl.ds(v * L, L)] = val

        def gather_start(g, b):
            pltpu.async_copy(table_hbm.at[idx_v.at[g]], x_v.at[b], gsems[b])

        def gather_wait(b):
            pltpu.make_async_copy(
                table_hbm.at[idx_v.at[0]], x_v.at[b], gsems[b]
            ).wait()

        def out_start(g, b):
            for ti in range(TI):
                pltpu.async_copy(
                    out_v.at[b, ti, pl.ds(0, 8), pl.ds(0, BLK)],
                    out_hbm.at[g, ti, wid],
                    osems[b],
                )

        def out_wait(b):
            for ti in range(TI):
                pltpu.make_async_copy(
                    out_v.at[b, ti, pl.ds(0, 8), pl.ds(0, BLK)],
                    out_hbm.at[0, ti, 0],
                    osems[b],
                ).wait()

        Ws = [wb_v[0, pl.ds(j * L, L)] for j in range(FV)]
        Bs = [wb_v[1, pl.ds(j * L, L)] for j in range(FV)]
        lane15 = lax.iota(jnp.int32, L) == jnp.int32(L - 1)
        dio = lax.iota(jnp.int32, L)
        chi = [lax.shift_right_arithmetic(dio + jnp.int32(16 * j), jnp.int32(3))
               for j in range(FV)]
        cmid = [(dio + jnp.int32(16 * j)) & jnp.int32(7) for j in range(FV)]

        def compute_chunk(g, b):
            ps = [pos_v[g, pl.ds(j * L, L)] for j in range(FV)]

            @plsc.parallel_loop(0, BLK, unroll=4)
            def _(i):
                xs = []
                for j in range(FV):
                    xs.append(x_v[b, i, pl.ds(j * L, L)] + ps[j])
                ssum = (xs[0] + xs[1]) + (xs[2] + xs[3])
                qs = [x * x for x in xs]
                qsum = (qs[0] + qs[1]) + (qs[2] + qs[3])
                for j in range(FV):
                    x_v[b, i, pl.ds(j * L, L)] = xs[j]
                sc = plsc.cumsum(ssum)
                qc = plsc.cumsum(qsum)
                iv = jnp.broadcast_to(i, (L,)).astype(jnp.int32)
                plsc.store_scatter(st_v.at[b, 0], [iv], sc, mask=lane15)
                plsc.store_scatter(st_v.at[b, 1], [iv], qc, mask=lane15)

            @plsc.parallel_loop(0, BLK // 16)
            def _(kg):
                r0 = kg * 16
                sv = st_v[b, 0, pl.ds(r0, L)]
                qv = st_v[b, 1, pl.ds(r0, L)]
                mean16 = sv * inv_emb
                var16 = qv * inv_emb - mean16 * mean16
                rstd16 = _rsqrt_newton(var16 + np.float32(LN_EPS))
                st_v[b, 0, pl.ds(r0, L)] = rstd16
                st_v[b, 1, pl.ds(r0, L)] = mean16 * rstd16

            @plsc.parallel_loop(0, BLK, unroll=4)
            def _(i):
                iv = jnp.broadcast_to(i, (L,)).astype(jnp.int32)
                rs = plsc.load_gather(st_v.at[b, 0], [iv])
                cm = plsc.load_gather(st_v.at[b, 1], [iv])
                for j in range(FV):
                    x = x_v[b, i, pl.ds(j * L, L)]
                    y = (x * rs - cm) * Ws[j] + Bs[j]
                    plsc.store_scatter(out_v.at[b], [chi[j], cmid[j], iv], y)

        for b0 in range(NBUF):
            gather_start(b0, b0)

        def ring_step(outer, _):
            for b in range(NBUF):
                g = outer * NBUF + b

                @pl.when(g < S)
                def _():
                    gather_wait(b)

                    @pl.when(g >= NBUF)
                    def _():
                        out_wait(b)

                    compute_chunk(g, b)

                    @pl.when(g + NBUF < S)
                    def _():
                        gather_start(g + NBUF, b)

                    out_start(g, b)
            return 0

        lax.fori_loop(0, (S + NBUF - 1) // NBUF, ring_step, 0)

        for b0 in range(NBUF):
            out_wait(b0)

    return k


@functools.lru_cache(maxsize=None)
def _kernel_fn(N, S, emb, interpret):
    return _build(N, S, emb, interpret)


def _call(src, emb_table, pos_table, ln_w, ln_b, interpret=False):
    N, S = src.shape
    emb = emb_table.shape[1]
    assert N % BLK == 0 and N // BLK == NW and emb % 16 == 0

    idx_t = src.astype(jnp.int32).reshape(NW, S, BLK)
    pos = pos_table[:S]
    wb = jnp.stack([ln_w, ln_b])
    fn = _kernel_fn(N, S, emb, interpret)
    out5 = fn(idx_t, emb_table, pos, wb)
    return out5.transpose(2, 4, 0, 1, 3).reshape(N, S, emb)


def kernel(src, emb_table, pos_table, ln_w, ln_b):
    return _call(src, emb_table, pos_table, ln_w, ln_b)


# speedup vs baseline: 1.1081x; 1.1081x over previous
import functools

import jax
import jax.numpy as jnp
import numpy as np
from jax import lax
from jax.experimental import pallas as pl
from jax.experimental.pallas import tpu as pltpu
from jax.experimental.pallas import tpu_sc as plsc

NC = 2
NS = 16
NW = NC * NS
L = 16
LN_EPS = 1e-5
BLK = 128
NBUF = 3


def _rsqrt_newton(x):
    i = lax.bitcast_convert_type(x, jnp.int32)
    i = jnp.int32(0x5F3759DF) - lax.shift_right_arithmetic(i, jnp.int32(1))
    y = lax.bitcast_convert_type(i, jnp.float32)
    half_x = jnp.float32(0.5) * x
    for _ in range(3):
        y = y * (jnp.float32(1.5) - half_x * y * y)
    return y


def _build(N, S, emb, interpret=False):
    FV = emb // L
    TI = emb // 8
    inv_emb = np.float32(1.0 / emb)
    nblk = N // BLK

    mesh = plsc.VectorSubcoreMesh(
        core_axis_name="c", subcore_axis_name="s", num_cores=NC, num_subcores=NS
    )

    @functools.partial(
        pl.kernel,
        out_type=jax.ShapeDtypeStruct((S, TI, nblk, 8, BLK), jnp.float32),
        mesh=mesh,
        scratch_types=[
            pltpu.VMEM((S, BLK), jnp.int32),
            pltpu.VMEM((S, BLK), jnp.int32),
            pltpu.VMEM((S, emb), jnp.float32),
            pltpu.VMEM((2, emb), jnp.float32),
            pltpu.VMEM((NBUF, BLK, 2 * emb), jnp.float32),
            pltpu.VMEM((NBUF, TI, 8, BLK + 1), jnp.float32),
            pltpu.VMEM((NBUF, 2, BLK), jnp.float32),
        ]
        + [pltpu.SemaphoreType.DMA] * (2 * NBUF),
        compiler_params=pltpu.CompilerParams(
            needs_layout_passes=False, use_tc_tiling_on_sc=False
        ),
        interpret=interpret,
    )
    def k(idx_hbm, table_hbm, pos_hbm, wb_hbm, out_hbm,
          idx_v, stage_v, pos_v, wb_v, x_v, out_v, st_v, *sems):
        gsems = sems[:NBUF]
        osems = sems[NBUF:]
        wid = lax.axis_index("s") * NC + lax.axis_index("c")

        pltpu.sync_copy(idx_hbm.at[wid], stage_v)
        pltpu.sync_copy(pos_hbm, pos_v)
        pltpu.sync_copy(wb_hbm, wb_v)

        tio = lax.iota(jnp.int32, L) * jnp.int32(S)

        @plsc.parallel_loop(0, S, unroll=2)
        def _(kk):
            for v in range(BLK // L):
                f = tio + (kk + jnp.int32(v * L * S))
                val = plsc.load_gather(
                    stage_v,
                    [lax.shift_right_arithmetic(f, jnp.int32(7)),
                     f & jnp.int32(BLK - 1)],
                )
                idx_v[kk, pl.ds(v * L, L)] = val

        def gather_start(g, b):
            pltpu.async_copy(table_hbm.at[idx_v.at[g]], x_v.at[b], gsems[b])

        def gather_wait(b):
            pltpu.make_async_copy(
                table_hbm.at[idx_v.at[0]], x_v.at[b], gsems[b]
            ).wait()

        def out_start(g, b):
            for ti in range(TI):
                pltpu.async_copy(
                    out_v.at[b, ti, pl.ds(0, 8), pl.ds(0, BLK)],
                    out_hbm.at[g, ti, wid],
                    osems[b],
                )

        def out_wait(b):
            for ti in range(TI):
                pltpu.make_async_copy(
                    out_v.at[b, ti, pl.ds(0, 8), pl.ds(0, BLK)],
                    out_hbm.at[0, ti, 0],
                    osems[b],
                ).wait()

        Ws = [wb_v[0, pl.ds(j * L, L)] for j in range(FV)]
        Bs = [wb_v[1, pl.ds(j * L, L)] for j in range(FV)]
        lane15 = lax.iota(jnp.int32, L) == jnp.int32(L - 1)
        dio = lax.iota(jnp.int32, L)
        chi = [lax.shift_right_arithmetic(dio + jnp.int32(16 * j), jnp.int32(3))
               for j in range(FV)]
        cmid = [(dio + jnp.int32(16 * j)) & jnp.int32(7) for j in range(FV)]

        def compute_chunk(g, b):
            ps = [pos_v[g, pl.ds(j * L, L)] for j in range(FV)]

            @plsc.parallel_loop(0, BLK, unroll=4)
            def _(i):
                xs = []
                for j in range(FV):
                    xs.append(x_v[b, i, pl.ds(j * L, L)] + ps[j])
                ssum = (xs[0] + xs[1]) + (xs[2] + xs[3])
                qs = [x * x for x in xs]
                qsum = (qs[0] + qs[1]) + (qs[2] + qs[3])
                for j in range(FV):
                    x_v[b, i, pl.ds(j * L, L)] = xs[j]
                sc = plsc.cumsum(ssum)
                qc = plsc.cumsum(qsum)
                iv = jnp.broadcast_to(i, (L,)).astype(jnp.int32)
                plsc.store_scatter(st_v.at[b, 0], [iv], sc, mask=lane15)
                plsc.store_scatter(st_v.at[b, 1], [iv], qc, mask=lane15)

            @plsc.parallel_loop(0, BLK // 16)
            def _(kg):
                r0 = kg * 16
                sv = st_v[b, 0, pl.ds(r0, L)]
                qv = st_v[b, 1, pl.ds(r0, L)]
                mean16 = sv * inv_emb
                var16 = qv * inv_emb - mean16 * mean16
                rstd16 = _rsqrt_newton(var16 + np.float32(LN_EPS))
                st_v[b, 0, pl.ds(r0, L)] = rstd16
                st_v[b, 1, pl.ds(r0, L)] = mean16 * rstd16

            @plsc.parallel_loop(0, BLK, unroll=4)
            def _(i):
                iv = jnp.broadcast_to(i, (L,)).astype(jnp.int32)
                rs = plsc.load_gather(st_v.at[b, 0], [iv])
                cm = plsc.load_gather(st_v.at[b, 1], [iv])
                for j in range(FV):
                    x = x_v[b, i, pl.ds(j * L, L)]
                    y = (x * rs - cm) * Ws[j] + Bs[j]
                    plsc.store_scatter(out_v.at[b], [chi[j], cmid[j], iv], y)

        for b0 in range(NBUF):
            gather_start(b0, b0)

        def ring_step(outer, _):
            for b in range(NBUF):
                g = outer * NBUF + b

                @pl.when(g < S)
                def _():
                    gather_wait(b)

                    @pl.when(g >= NBUF)
                    def _():
                        out_wait(b)

                    compute_chunk(g, b)

                    @pl.when(g + NBUF < S)
                    def _():
                        gather_start(g + NBUF, b)

                    out_start(g, b)
            return 0

        lax.fori_loop(0, (S + NBUF - 1) // NBUF, ring_step, 0)

        for b0 in range(NBUF):
            out_wait(b0)

    return k


@functools.lru_cache(maxsize=None)
def _kernel_fn(N, S, emb, interpret):
    return _build(N, S, emb, interpret)


def _call(src, emb_table, pos_table, ln_w, ln_b, interpret=False):
    N, S = src.shape
    emb = emb_table.shape[1]
    assert N % BLK == 0 and N // BLK == NW and emb % 16 == 0

    idx_t = src.astype(jnp.int32).reshape(NW, S, BLK)
    emb_table = jnp.pad(emb_table, ((0, (-emb_table.shape[0]) % 8), (0, emb)))
    pos = pos_table[:S]
    wb = jnp.stack([ln_w, ln_b])
    fn = _kernel_fn(N, S, emb, interpret)
    out5 = fn(idx_t, emb_table, pos, wb)
    return out5.transpose(2, 4, 0, 1, 3).reshape(N, S, emb)


def kernel(src, emb_table, pos_table, ln_w, ln_b):
    return _call(src, emb_table, pos_table, ln_w, ln_b)
